# baseline (speedup 1.0000x reference)
import functools

import jax
import jax.numpy as jnp
from jax import lax
from jax.experimental import pallas as pl
from jax.experimental.pallas import tpu as pltpu
from jax.experimental.pallas import tpu_sc as plsc

NUM_CORES = 2
NUM_SUBCORES = 16
NW = NUM_CORES * NUM_SUBCORES

BATCH = 16384
HIST = 50
DIM = 64
TOTAL = BATCH * HIST
PER_W = TOTAL // NW
B_PER_W = BATCH // NW
BCHUNK = 8
CHUNK = BCHUNK * HIST
NCHUNK = B_PER_W // BCHUNK
NBUF = 3
NOUT = NCHUNK // NBUF
NTAIL = NCHUNK - NOUT * NBUF

_mesh = plsc.VectorSubcoreMesh(
    core_axis_name="c", subcore_axis_name="s",
    num_cores=NUM_CORES, num_subcores=NUM_SUBCORES,
)


@functools.partial(
    pl.kernel,
    out_type=jax.ShapeDtypeStruct((BATCH, HIST, DIM), jnp.float32),
    mesh=_mesh,
    scratch_types=[
        pltpu.VMEM((PER_W,), jnp.int32),
        *[pltpu.VMEM((CHUNK, DIM), jnp.float32) for _ in range(NBUF)],
        *[pltpu.SemaphoreType.DMA for _ in range(NBUF)],
        *[pltpu.SemaphoreType.DMA for _ in range(NBUF)],
    ],
    compiler_params=pltpu.CompilerParams(use_tc_tiling_on_sc=False),
)
def _embed_sc(idx_hbm, table_hbm, out_hbm, idx_v, *scratch):
    bufs = scratch[:NBUF]
    gsem = scratch[NBUF:2 * NBUF]
    osem = scratch[2 * NBUF:]

    wid = lax.axis_index("s") * NUM_CORES + lax.axis_index("c")
    b_base = wid * B_PER_W
    pltpu.sync_copy(idx_hbm.at[wid], idx_v)

    def fire_gather(j, buf, sem):
        pltpu.async_copy(
            table_hbm.at[idx_v.at[pl.ds(j * CHUNK, CHUNK)]], buf, sem)

    def wait_gather(buf, sem):
        pltpu.make_async_copy(
            table_hbm.at[pl.ds(0, CHUNK)], buf, sem).wait()

    def fire_writeback(j, buf, sem):
        for k in range(BCHUNK):
            pltpu.async_copy(
                buf.at[pl.ds(k * HIST, HIST)],
                out_hbm.at[b_base + j * BCHUNK + k], sem)

    def wait_writeback(buf, sem):
        for k in range(BCHUNK):
            pltpu.make_async_copy(
                buf.at[pl.ds(0, HIST)], out_hbm.at[b_base], sem).wait()

    for b in range(NBUF):
        fire_gather(b, bufs[b], gsem[b])

    def body(t, carry):
        j0 = t * NBUF
        for b in range(NBUF):
            j = j0 + b
            wait_gather(bufs[b], gsem[b])
            fire_writeback(j, bufs[b], osem[b])

            @pl.when(j + NBUF < NCHUNK)
            def _():
                wait_writeback(bufs[b], osem[b])
                fire_gather(j + NBUF, bufs[b], gsem[b])
        return carry

    lax.fori_loop(0, NOUT, body, 0)
    for b in range(NTAIL):
        j = NOUT * NBUF + b
        wait_gather(bufs[b], gsem[b])
        fire_writeback(j, bufs[b], osem[b])
    for b in range(NBUF):
        wait_writeback(bufs[b], osem[b])


def kernel(input_ids, weight):
    idx = input_ids.reshape(NW, PER_W).astype(jnp.int32)
    return _embed_sc(idx, weight)
